# Initial kernel scaffold; baseline (speedup 1.0000x reference)
import jax
import jax.numpy as jnp
from jax.experimental import pallas as pl


def kernel(x, inter):
    raise NotImplementedError("write your pallas kernel here")


# speedup vs baseline: 191.3905x; 191.3905x over previous
import functools

import jax
import jax.numpy as jnp
from jax.experimental import pallas as pl
from jax.experimental.pallas import tpu as pltpu


def _body(x_ref, inter_ref, o_ref, *, inv_cnt):
    x = x_ref[0]
    m = jnp.sum(x, axis=0, keepdims=True) * inv_cnt
    it = inter_ref[...]
    blend = x * (1.0 - it) + m * it
    o_ref[0] = jnp.where(blend > 0, x, 0.0)


def kernel(x, inter):
    B, C, H, W = x.shape
    HW = H * W
    x3 = x.reshape(B, C, HW)
    it2 = inter.reshape(C, HW)
    inv_cnt = 1.0 / (C + 1e-10)
    out = pl.pallas_call(
        functools.partial(_body, inv_cnt=inv_cnt),
        grid=(B,),
        in_specs=[
            pl.BlockSpec((1, C, HW), lambda b: (b, 0, 0)),
            pl.BlockSpec((C, HW), lambda b: (0, 0)),
        ],
        out_specs=pl.BlockSpec((1, C, HW), lambda b: (b, 0, 0)),
        out_shape=jax.ShapeDtypeStruct((B, C, HW), x.dtype),
        compiler_params=pltpu.CompilerParams(
            dimension_semantics=("arbitrary",),
        ),
    )(x3, it2)
    return out.reshape(B, C, H, W)


# speedup vs baseline: 218.9086x; 1.1438x over previous
import functools

import jax
import jax.numpy as jnp
from jax.experimental import pallas as pl
from jax.experimental.pallas import tpu as pltpu


_BB = 4


def _body(x_ref, inter_ref, o_ref, *, inv_cnt):
    it = inter_ref[...]
    for i in range(_BB):
        x = x_ref[i]
        m = jnp.sum(x, axis=0, keepdims=True) * inv_cnt
        blend = x * (1.0 - it) + m * it
        o_ref[i] = jnp.where(blend > 0, x, 0.0)


def kernel(x, inter):
    B, C, H, W = x.shape
    HW = H * W
    x3 = x.reshape(B, C, HW)
    it2 = inter.reshape(C, HW)
    inv_cnt = 1.0 / (C + 1e-10)
    out = pl.pallas_call(
        functools.partial(_body, inv_cnt=inv_cnt),
        grid=(B // _BB,),
        in_specs=[
            pl.BlockSpec((_BB, C, HW), lambda b: (b, 0, 0)),
            pl.BlockSpec((C, HW), lambda b: (0, 0)),
        ],
        out_specs=pl.BlockSpec((_BB, C, HW), lambda b: (b, 0, 0)),
        out_shape=jax.ShapeDtypeStruct((B, C, HW), x.dtype),
        compiler_params=pltpu.CompilerParams(
            dimension_semantics=("parallel",),
        ),
    )(x3, it2)
    return out.reshape(B, C, H, W)
